# baseline (speedup 1.0000x reference)
import functools

import jax
import jax.numpy as jnp
from jax import lax
from jax.experimental import pallas as pl
from jax.experimental.pallas import tpu as pltpu
from jax.experimental.pallas import tpu_sc as plsc

_CHUNK = 64
_NW = 32
_NB = 5
_NG = 2


@functools.cache
def _build(B, D, rows_per_w):
    mesh = plsc.VectorSubcoreMesh(core_axis_name="c", subcore_axis_name="s")
    n_iters = rows_per_w // (_NB * _NG)

    @functools.partial(
        pl.kernel,
        mesh=mesh,
        out_type=jax.ShapeDtypeStruct((B, D), jnp.float32),
        scratch_types=[
            pltpu.VMEM((rows_per_w, _CHUNK), jnp.int32),
            pltpu.VMEM((_NG * _NB, _CHUNK, D), jnp.float32),
            pltpu.VMEM_SHARED((16, 2, _CHUNK, D), jnp.float32),
        ]
        + [pltpu.SemaphoreType.DMA] * (4 * _NB),
    )
    def k(idx_hbm, table_hbm, out_hbm, idx_v, rows_v, sp_v, *sems):
        gsem = sems[: 2 * _NB]
        osem = sems[2 * _NB:]
        sid = lax.axis_index("s")
        wid = sid * 2 + lax.axis_index("c")
        row0 = wid * rows_per_w
        pltpu.sync_copy(idx_hbm.at[wid], idx_v)

        def body(it, carry):
            j0 = it * _NG * _NB
            sh = []
            gh = [
                pltpu.async_copy(
                    table_hbm.at[idx_v.at[j0 + b]], rows_v.at[b], gsem[b]
                )
                for b in range(_NB)
            ]
            for b in range(_NB):
                gh[b].wait()
                sh.append(
                    pltpu.async_copy(
                        rows_v.at[b],
                        out_hbm.at[pl.ds((row0 + j0 + b) * _CHUNK, _CHUNK)],
                        osem[b],
                    )
                )
            j1 = j0 + _NB
            gh = [
                pltpu.async_copy(
                    table_hbm.at[idx_v.at[j1 + b]],
                    rows_v.at[_NB + b],
                    gsem[_NB + b],
                )
                for b in range(_NB)
            ]
            sh1 = []
            for b in range(_NB):
                gh[b].wait()
                if b >= 2:
                    sh1[b - 2].wait()
                pltpu.async_copy(
                    rows_v.at[_NB + b], sp_v.at[sid, b % 2], gsem[_NB + b]
                ).wait()
                sh1.append(
                    pltpu.async_copy(
                        sp_v.at[sid, b % 2],
                        out_hbm.at[pl.ds((row0 + j1 + b) * _CHUNK, _CHUNK)],
                        osem[_NB + b],
                    )
                )
            sh.extend(sh1[-2:])
            for h in sh:
                h.wait()
            return carry

        lax.fori_loop(0, n_iters, body, 0)

    return k


def kernel(labels, table):
    D = table.shape[1]
    BT, T = labels.shape
    B = BT * T
    idx = labels.astype(jnp.int32).T
    n_rows = B // _CHUNK
    rows_per_w = n_rows // _NW
    idx3 = idx.reshape(_NW, rows_per_w, _CHUNK)
    out = _build(B, D, rows_per_w)(idx3, table)
    return out.reshape(T, BT, D).transpose(1, 0, 2)


# speedup vs baseline: 1.1030x; 1.1030x over previous
import functools

import jax
import jax.numpy as jnp
from jax import lax
from jax.experimental import pallas as pl
from jax.experimental.pallas import tpu as pltpu
from jax.experimental.pallas import tpu_sc as plsc

_CHUNK = 64
_NW = 32
_NB = 5
_NG = 2


@functools.cache
def _build(B, D, rows_per_w):
    mesh = plsc.VectorSubcoreMesh(core_axis_name="c", subcore_axis_name="s")
    n_iters = rows_per_w // (_NB * _NG)

    @functools.partial(
        pl.kernel,
        mesh=mesh,
        out_type=jax.ShapeDtypeStruct((B, D), jnp.float32),
        scratch_types=[
            pltpu.VMEM((rows_per_w, _CHUNK), jnp.int32),
            pltpu.VMEM((_NG * _NB, _CHUNK, D), jnp.float32),
        ]
        + [pltpu.SemaphoreType.DMA] * (2 * _NG * _NB),
    )
    def k(idx_hbm, table_hbm, out_hbm, idx_v, rows_v, *sems):
        gsem = sems[: _NG * _NB]
        osem = sems[_NG * _NB:]
        wid = lax.axis_index("s") * 2 + lax.axis_index("c")
        row0 = wid * rows_per_w
        pltpu.sync_copy(idx_hbm.at[wid], idx_v)

        def do_group(jr, half, pre_drain):
            off = half * _NB
            if pre_drain:
                for b in range(_NB):
                    pltpu.make_async_copy(
                        rows_v.at[off + b],
                        out_hbm.at[
                            pl.ds((row0 + jr - _NG * _NB + b) * _CHUNK, _CHUNK)
                        ],
                        osem[off + b],
                    ).wait()
            gh = [
                pltpu.async_copy(
                    table_hbm.at[idx_v.at[jr + b]],
                    rows_v.at[off + b],
                    gsem[off + b],
                )
                for b in range(_NB)
            ]
            for b in range(_NB):
                gh[b].wait()
                pltpu.async_copy(
                    rows_v.at[off + b],
                    out_hbm.at[pl.ds((row0 + jr + b) * _CHUNK, _CHUNK)],
                    osem[off + b],
                )

        for r in range(_NG):
            do_group(r * _NB, r, pre_drain=False)

        def body(it, carry):
            j0 = (it + 1) * _NG * _NB
            for r in range(_NG):
                do_group(j0 + r * _NB, r, pre_drain=True)
            return carry

        lax.fori_loop(0, n_iters - 1, body, 0)

        jlast = (n_iters - 1) * _NG * _NB
        for r in range(_NG):
            for b in range(_NB):
                pltpu.make_async_copy(
                    rows_v.at[r * _NB + b],
                    out_hbm.at[
                        pl.ds((row0 + jlast + r * _NB + b) * _CHUNK, _CHUNK)
                    ],
                    osem[r * _NB + b],
                ).wait()

    return k


def kernel(labels, table):
    D = table.shape[1]
    BT, T = labels.shape
    B = BT * T
    idx = labels.astype(jnp.int32).T
    n_rows = B // _CHUNK
    rows_per_w = n_rows // _NW
    idx3 = idx.reshape(_NW, rows_per_w, _CHUNK)
    out = _build(B, D, rows_per_w)(idx3, table)
    return out.reshape(T, BT, D).transpose(1, 0, 2)
